# baseline (speedup 1.0000x reference)
import jax
import jax.numpy as jnp
from jax import lax
from jax.experimental import pallas as pl

_NMS_THRESH = 0.5
_SCORE_THRESH = 0.01
_N = 5000
_ROWS = 8
_COLS = 640
_NP = _ROWS * _COLS
_NEG = -3.4e38


def _bmax(x):
    return jnp.max(x, axis=(0, 1), keepdims=True)


def _nms_body(x1_ref, y1_ref, x2_ref, y2_ref, s_ref, keep_ref):
    x1 = x1_ref[...]
    y1 = y1_ref[...]
    x2 = x2_ref[...]
    y2 = y2_ref[...]
    s = s_ref[...]
    idxf = (lax.broadcasted_iota(jnp.int32, (_ROWS, _COLS), 0) * _COLS
            + lax.broadcasted_iota(jnp.int32, (_ROWS, _COLS), 1)
            ).astype(jnp.float32)
    area = (x2 - x1) * (y2 - y1)

    def argmax(ms):
        maxv = _bmax(ms)
        mf = -_bmax(jnp.where(ms == maxv, -idxf, _NEG))
        return maxv, mf

    def iou_vs(onehot):
        x1m = _bmax(jnp.where(onehot, x1, _NEG))
        y1m = _bmax(jnp.where(onehot, y1, _NEG))
        x2m = _bmax(jnp.where(onehot, x2, _NEG))
        y2m = _bmax(jnp.where(onehot, y2, _NEG))
        aream = (x2m - x1m) * (y2m - y1m)
        w = jnp.maximum(0.0, jnp.minimum(x2, x2m) - jnp.maximum(x1, x1m))
        h = jnp.maximum(0.0, jnp.minimum(y2, y2m) - jnp.maximum(y1, y1m))
        inter = w * h
        return inter / (area + aream - inter)

    def pair_round(carry):
        ms, keep, maxv, mf = carry
        onehot1 = idxf == mf
        iou1 = iou_vs(onehot1)
        dead1 = (iou1 > _NMS_THRESH) | onehot1
        ms1 = jnp.where(onehot1, -1.0, ms)
        maxv2, mf2 = argmax(ms1)
        live2 = maxv2 >= 0.0
        onehot2 = idxf == mf2
        iou2 = iou_vs(onehot2)
        dead2 = (iou2 > _NMS_THRESH) | onehot2
        sup2 = _bmax(jnp.where(onehot2 & dead1, 1.0, 0.0))
        valid2 = live2 & (sup2 < 0.5)
        dead = dead1 | (dead2 & valid2)
        keep = jnp.where(onehot1 | (onehot2 & valid2), 1.0, keep)
        ms = jnp.where(dead, -1.0, ms)
        maxv3, mf3 = argmax(ms)
        return ms, keep, maxv3, mf3

    def cond(carry):
        return carry[2][0, 0] >= _SCORE_THRESH

    ms0 = jnp.where(idxf < float(_N), s, -1.0)
    keep0 = jnp.zeros((_ROWS, _COLS), dtype=jnp.float32)
    maxv0, m0 = argmax(ms0)
    _, keep, _, _ = lax.while_loop(cond, pair_round, (ms0, keep0, maxv0, m0))
    keep_ref[...] = jnp.where(s >= _SCORE_THRESH, keep, 0.0)


def _pad2d(v):
    return jnp.pad(v, (0, _NP - _N)).reshape(_ROWS, _COLS)


@jax.jit
def kernel(boxes, scores):
    cols2d = [_pad2d(boxes[:, i]) for i in range(4)]
    svec = _pad2d(scores)
    keep2d = pl.pallas_call(
        _nms_body,
        in_specs=[pl.BlockSpec((_ROWS, _COLS), lambda: (0, 0))] * 5,
        out_specs=pl.BlockSpec((_ROWS, _COLS), lambda: (0, 0)),
        out_shape=jax.ShapeDtypeStruct((_ROWS, _COLS), jnp.float32),
    )(*cols2d, svec)
    keep = keep2d.reshape(_NP)[:_N]
    kept_boxes = boxes * keep[:, None]
    kept_scores = scores * keep
    return jnp.concatenate([kept_boxes, kept_scores[:, None]], axis=1)


# speedup vs baseline: 1.0164x; 1.0164x over previous
import jax
import jax.numpy as jnp
from jax import lax
from jax.experimental import pallas as pl
from jax.experimental.pallas import tpu as pltpu

_NMS_THRESH = 0.5
_SCORE_THRESH = 0.01
_N = 5000
_ROWS = 8
_COLS = 640
_NP = _ROWS * _COLS


def _nms_body(x1s, y1s, x2s, y2s, x1_ref, y1_ref, x2_ref, y2_ref, s_ref,
              keep_ref):
    x1 = x1_ref[...]
    y1 = y1_ref[...]
    x2 = x2_ref[...]
    y2 = y2_ref[...]
    s = s_ref[...]
    idxf = (lax.broadcasted_iota(jnp.int32, (_ROWS, _COLS), 0) * _COLS
            + lax.broadcasted_iota(jnp.int32, (_ROWS, _COLS), 1)
            ).astype(jnp.float32)

    def argmax(ms):
        maxv = jnp.max(ms)
        mf = jnp.min(jnp.where(ms == maxv, idxf, float(_NP)))
        return maxv, mf

    area = (x2 - x1) * (y2 - y1)

    def iou_vs(m):
        x1m = x1s[m]
        y1m = y1s[m]
        x2m = x2s[m]
        y2m = y2s[m]
        aream = (x2m - x1m) * (y2m - y1m)
        w = jnp.maximum(0.0, jnp.minimum(x2, x2m) - jnp.maximum(x1, x1m))
        h = jnp.maximum(0.0, jnp.minimum(y2, y2m) - jnp.maximum(y1, y1m))
        inter = w * h
        return inter / (area + aream - inter)

    def pair_round(carry):
        ms, keep, maxv, mf = carry
        live1 = maxv >= 0.0
        m1 = jnp.minimum(mf, float(_NP - 1)).astype(jnp.int32)
        onehot1 = (idxf == mf) & live1
        iou1 = iou_vs(m1)
        dead1 = ((iou1 > _NMS_THRESH) & live1) | onehot1
        ms1 = jnp.where(onehot1, -1.0, ms)
        maxv2, mf2 = argmax(ms1)
        live2 = maxv2 >= 0.0
        m2 = jnp.minimum(mf2, float(_NP - 1)).astype(jnp.int32)
        onehot2 = idxf == mf2
        iou2 = iou_vs(m2)
        dead2 = (iou2 > _NMS_THRESH) | onehot2
        sup2 = jnp.max(jnp.where(onehot2 & dead1, 1.0, 0.0))
        valid2 = live2 & (sup2 < 0.5)
        dead = dead1 | (dead2 & valid2)
        keep = jnp.where(onehot1 | (onehot2 & valid2), 1.0, keep)
        ms = jnp.where(dead, -1.0, ms)
        maxv3, mf3 = argmax(ms)
        return ms, keep, maxv3, mf3

    def cond(carry):
        return carry[2] >= _SCORE_THRESH

    def body(carry):
        return pair_round(pair_round(carry))

    ms0 = jnp.where(idxf < float(_N), s, -1.0)
    keep0 = jnp.zeros((_ROWS, _COLS), dtype=jnp.float32)
    maxv0, m0 = argmax(ms0)
    _, keep, _, _ = lax.while_loop(cond, body, (ms0, keep0, maxv0, m0))
    keep_ref[...] = jnp.where(s >= _SCORE_THRESH, keep, 0.0)


def _pad(v):
    return jnp.pad(v, (0, _NP - _N))


@jax.jit
def kernel(boxes, scores):
    flat = [_pad(boxes[:, i]) for i in range(4)]
    cols2d = [v.reshape(_ROWS, _COLS) for v in flat]
    svec = _pad(scores).reshape(_ROWS, _COLS)
    smem_spec = pl.BlockSpec(memory_space=pltpu.SMEM)
    keep2d = pl.pallas_call(
        _nms_body,
        in_specs=[smem_spec] * 4 + [pl.BlockSpec((_ROWS, _COLS),
                                                 lambda: (0, 0))] * 5,
        out_specs=pl.BlockSpec((_ROWS, _COLS), lambda: (0, 0)),
        out_shape=jax.ShapeDtypeStruct((_ROWS, _COLS), jnp.float32),
    )(*flat, *cols2d, svec)
    keep = keep2d.reshape(_NP)[:_N]
    kept_boxes = boxes * keep[:, None]
    kept_scores = scores * keep
    return jnp.concatenate([kept_boxes, kept_scores[:, None]], axis=1)


# speedup vs baseline: 1.0384x; 1.0217x over previous
import jax
import jax.numpy as jnp
from jax import lax
from jax.experimental import pallas as pl
from jax.experimental.pallas import tpu as pltpu

_NMS_THRESH = 0.5
_SCORE_THRESH = 0.01
_N = 5000
_ROWS = 8
_COLS = 640
_NP = _ROWS * _COLS


def _nms_body(x1s, y1s, x2s, y2s, x1_ref, y1_ref, x2_ref, y2_ref, s_ref,
              keep_ref):
    x1 = x1_ref[...]
    y1 = y1_ref[...]
    x2 = x2_ref[...]
    y2 = y2_ref[...]
    s = s_ref[...]
    idxf = (lax.broadcasted_iota(jnp.int32, (_ROWS, _COLS), 0) * _COLS
            + lax.broadcasted_iota(jnp.int32, (_ROWS, _COLS), 1)
            ).astype(jnp.float32)

    def argmax(ms):
        maxv = jnp.max(ms)
        mf = jnp.min(jnp.where(ms == maxv, idxf, float(_NP)))
        return maxv, mf

    area = (x2 - x1) * (y2 - y1)

    def iou_vs(m):
        x1m = x1s[m]
        y1m = y1s[m]
        x2m = x2s[m]
        y2m = y2s[m]
        aream = (x2m - x1m) * (y2m - y1m)
        w = jnp.maximum(0.0, jnp.minimum(x2, x2m) - jnp.maximum(x1, x1m))
        h = jnp.maximum(0.0, jnp.minimum(y2, y2m) - jnp.maximum(y1, y1m))
        inter = w * h
        return inter / (area + aream - inter)

    def pair_round(carry):
        ms, keep, maxv, mf = carry
        m1 = mf.astype(jnp.int32)
        onehot1 = idxf == mf
        iou1 = iou_vs(m1)
        dead1 = (iou1 > _NMS_THRESH) | onehot1
        ms1 = jnp.where(onehot1, -1.0, ms)
        maxv2, mf2 = argmax(ms1)
        live2 = maxv2 >= 0.0
        m2 = jnp.minimum(mf2, float(_NP - 1)).astype(jnp.int32)
        onehot2 = idxf == mf2
        iou2 = iou_vs(m2)
        dead2 = (iou2 > _NMS_THRESH) | onehot2
        sup2 = jnp.max(jnp.where(onehot2 & dead1, 1.0, 0.0))
        valid2 = live2 & (sup2 < 0.5)
        dead = dead1 | (dead2 & valid2)
        keep = jnp.where(onehot1 | (onehot2 & valid2), 1.0, keep)
        ms = jnp.where(dead, -1.0, ms)
        maxv3, mf3 = argmax(ms)
        return ms, keep, maxv3, mf3

    def cond(carry):
        return carry[2] >= _SCORE_THRESH

    body = pair_round

    ms0 = jnp.where(idxf < float(_N), s, -1.0)
    keep0 = jnp.zeros((_ROWS, _COLS), dtype=jnp.float32)
    maxv0, m0 = argmax(ms0)
    _, keep, _, _ = lax.while_loop(cond, body, (ms0, keep0, maxv0, m0))
    keep_ref[...] = jnp.where(s >= _SCORE_THRESH, keep, 0.0)


def _pad(v):
    return jnp.pad(v, (0, _NP - _N))


@jax.jit
def kernel(boxes, scores):
    flat = [_pad(boxes[:, i]) for i in range(4)]
    cols2d = [v.reshape(_ROWS, _COLS) for v in flat]
    svec = _pad(scores).reshape(_ROWS, _COLS)
    smem_spec = pl.BlockSpec(memory_space=pltpu.SMEM)
    keep2d = pl.pallas_call(
        _nms_body,
        in_specs=[smem_spec] * 4 + [pl.BlockSpec((_ROWS, _COLS),
                                                 lambda: (0, 0))] * 5,
        out_specs=pl.BlockSpec((_ROWS, _COLS), lambda: (0, 0)),
        out_shape=jax.ShapeDtypeStruct((_ROWS, _COLS), jnp.float32),
    )(*flat, *cols2d, svec)
    keep = keep2d.reshape(_NP)[:_N]
    kept_boxes = boxes * keep[:, None]
    kept_scores = scores * keep
    return jnp.concatenate([kept_boxes, kept_scores[:, None]], axis=1)
